# Initial kernel scaffold; baseline (speedup 1.0000x reference)
import jax
import jax.numpy as jnp
from jax.experimental import pallas as pl


def kernel(x, edge_index, edge_attr, batch, Wl1, bl1, Wr1, br1, We1, att1, bias1, bn1_g, bn1_b, Wl2, bl2, Wr2, br2, We2, att2, bias2, bn2_g, bn2_b, fc1_w, fc1_b, fc2_w, fc2_b, fc3_w, fc3_b):
    raise NotImplementedError("write your pallas kernel here")


# speedup vs baseline: 1.0002x; 1.0002x over previous
import jax
import jax.numpy as jnp
from jax.experimental import pallas as pl

N = 50000
B = 32
H1, C1 = 4, 32
H2, C2 = 4, 64


def _gatv2_layer(x, src, dst, ea, Wl, bl, Wr, br, We, att, bias, H, C):
    n = x.shape[0]
    xl = (x @ Wl + bl).reshape(n, H, C)
    xr = (x @ Wr + br).reshape(n, H, C)
    e = (ea @ We).reshape(-1, H, C)
    m = jax.nn.leaky_relu(xl[src] + xr[dst] + e, 0.01)
    alpha = jnp.sum(m * att[None, :, :], axis=-1)
    amax = jax.ops.segment_max(alpha, dst, num_segments=n)
    amax = jnp.where(jnp.isfinite(amax), amax, 0.0)
    ex = jnp.exp(alpha - amax[dst])
    denom = jax.ops.segment_sum(ex, dst, num_segments=n)
    a = ex / (denom[dst] + 1e-16)
    out = jax.ops.segment_sum(xl[src] * a[:, :, None], dst, num_segments=n)
    return out.reshape(n, H * C) + bias


def _bn(h, g, b):
    mu = jnp.mean(h, axis=0)
    var = jnp.var(h, axis=0)
    return (h - mu) / jnp.sqrt(var + 1e-5) * g + b


def _mlp_head_kernel(h_ref, w1_ref, b1_ref, w2_ref, b2_ref, w3_ref, b3_ref, o_ref):
    h = h_ref[...]
    h = jax.nn.leaky_relu(h @ w1_ref[...] + b1_ref[...], 0.01)
    h = jax.nn.leaky_relu(h @ w2_ref[...] + b2_ref[...], 0.01)
    o_ref[...] = h @ w3_ref[...] + b3_ref[...]


def kernel(x, edge_index, edge_attr, batch, Wl1, bl1, Wr1, br1, We1, att1, bias1, bn1_g, bn1_b, Wl2, bl2, Wr2, br2, We2, att2, bias2, bn2_g, bn2_b, fc1_w, fc1_b, fc2_w, fc2_b, fc3_w, fc3_b):
    n = x.shape[0]
    loop = jnp.arange(n, dtype=edge_index.dtype)
    src = jnp.concatenate([edge_index[0], loop])
    dst = jnp.concatenate([edge_index[1], loop])
    ea_mean = jnp.mean(edge_attr, axis=0, keepdims=True)
    ea = jnp.concatenate([edge_attr, jnp.broadcast_to(ea_mean, (n, edge_attr.shape[1]))], axis=0)
    h = _gatv2_layer(x, src, dst, ea, Wl1, bl1, Wr1, br1, We1, att1, bias1, H1, C1)
    h = jax.nn.leaky_relu(_bn(h, bn1_g, bn1_b), 0.01)
    h = _gatv2_layer(h, src, dst, ea, Wl2, bl2, Wr2, br2, We2, att2, bias2, H2, C2)
    h = jax.nn.leaky_relu(_bn(h, bn2_g, bn2_b), 0.01)
    s = jax.ops.segment_sum(h, batch, num_segments=B)
    cnt = jax.ops.segment_sum(jnp.ones((n,), jnp.float32), batch, num_segments=B)
    h = s / jnp.maximum(cnt, 1.0)[:, None]
    out = pl.pallas_call(
        _mlp_head_kernel,
        out_shape=jax.ShapeDtypeStruct((B, 1), jnp.float32),
    )(h, fc1_w, fc1_b, fc2_w, fc2_b, fc3_w, fc3_b)
    return out


# speedup vs baseline: 11.5815x; 11.5789x over previous
import functools

import jax
import jax.numpy as jnp
from jax import lax
from jax.experimental import pallas as pl
from jax.experimental.pallas import tpu as pltpu
from jax.experimental.pallas import tpu_sc as plsc

N = 50000
B = 32
H1, C1 = 4, 32
H2, C2 = 4, 64

NP = 50048
RPT = NP // 16
E_RAW = 800000
EP = 851968
CH = 128
NCHUNK = EP // (32 * CH)
ZROWS = 391

_MESH = plsc.VectorSubcoreMesh(core_axis_name="c", subcore_axis_name="s")
_SC_PARAMS = pltpu.CompilerParams(
    needs_layout_passes=False, use_tc_tiling_on_sc=False)


def _wid():
    return lax.axis_index("s") * 2 + lax.axis_index("c")


def _make_s1(H, C):
    D = H * C
    CPH = C // 16

    def body(xl, xr, srcr, dstr, ear, wear, attr, zr4,
             exo, deno,
             src_v, dst_v, ea_v, xlr, xrr, exb, exb16, w_v, a_v, zb4, den_sh,
             sem1, sem2):
        c = lax.axis_index("c")
        s = lax.axis_index("s")
        wid = _wid()
        pltpu.sync_copy(wear, w_v)
        pltpu.sync_copy(attr, a_v)
        pltpu.sync_copy(zr4, zb4)
        for k in range(8):
            pltpu.sync_copy(
                zb4, den_sh.at[pl.ds(s * RPT + k * ZROWS, ZROWS)])

        iota0 = lax.iota(jnp.int32, 16)

        def zero_row2(g, zcarry):
            p = iota0 + g * 16
            plsc.store_scatter(exb16, [p >> 4, p & 15],
                               jnp.zeros((16,), jnp.float32))
            return zcarry

        lax.fori_loop(0, (CH * 16) // 16, zero_row2, 0)
        plsc.subcore_barrier()
        iota = lax.iota(jnp.int32, 16)
        lane15 = iota == 15

        def chunk_body(t, carry):
            base = (wid * NCHUNK + t) * CH
            pltpu.sync_copy(srcr.at[pl.ds(base, CH)], src_v)
            pltpu.sync_copy(dstr.at[pl.ds(base, CH)], dst_v)
            pltpu.sync_copy(ear.at[pl.ds(base, CH)], ea_v)
            cp1 = pltpu.async_copy(xl.at[src_v], xlr, sem1)
            cp2 = pltpu.async_copy(xr.at[dst_v], xrr, sem2)
            cp1.wait()
            cp2.wait()

            def edge_body(e, ecarry):
                ea16 = plsc.load_gather(ea_v, [jnp.broadcast_to(e, (16,))])
                for h in range(H):
                    acc = jnp.zeros((16,), jnp.float32)
                    for j2 in range(CPH):
                        j = h * CPH + j2
                        zl = xlr[e, pl.ds(j * 16, 16)]
                        zrv = xrr[e, pl.ds(j * 16, 16)]
                        wv = w_v[pl.ds(j * 16, 16)]
                        av = a_v[pl.ds(j * 16, 16)]
                        z = zl + zrv + ea16 * wv
                        m = jnp.maximum(z, 0.01 * z)
                        acc = acc + m * av
                    csum = plsc.cumsum(acc)
                    plsc.store_scatter(
                        exb,
                        [jnp.broadcast_to(e, (16,)),
                         jnp.full((16,), h, jnp.int32)],
                        csum, mask=lane15)
                return ecarry

            lax.fori_loop(0, CH, edge_body, 0)
            for v in range((CH * H) // 16):
                p = iota + v * 16
                r = p >> 2
                cc = p & 3
                val = plsc.load_gather(exb, [r, cc])
                ev = jnp.exp(val)
                plsc.store_scatter(exb, [r, cc], ev)
                plsc.store_scatter(exb16, [r, cc], ev)
            pltpu.sync_copy(exb, exo.at[pl.ds(base, CH)])
            pltpu.sync_copy(exb16, den_sh.at[dst_v], add=True)
            return carry

        lax.fori_loop(0, NCHUNK, chunk_body, 0)
        plsc.subcore_barrier()
        pltpu.sync_copy(den_sh.at[pl.ds(s * RPT, RPT)],
                        deno.at[pl.ds(c * NP + s * RPT, RPT)])

    return pl.kernel(
        body,
        out_type=(
            jax.ShapeDtypeStruct((EP, H), jnp.float32),
            jax.ShapeDtypeStruct((2 * NP, 16), jnp.float32),
        ),
        mesh=_MESH,
        compiler_params=_SC_PARAMS,
        scratch_types=(
            pltpu.VMEM((CH,), jnp.int32),
            pltpu.VMEM((CH,), jnp.int32),
            pltpu.VMEM((CH,), jnp.float32),
            pltpu.VMEM((CH, D), jnp.float32),
            pltpu.VMEM((CH, D), jnp.float32),
            pltpu.VMEM((CH, H), jnp.float32),
            pltpu.VMEM((CH, 16), jnp.float32),
            pltpu.VMEM((D,), jnp.float32),
            pltpu.VMEM((D,), jnp.float32),
            pltpu.VMEM((ZROWS, 16), jnp.float32),
            pltpu.VMEM_SHARED((NP, 16), jnp.float32),
            pltpu.SemaphoreType.DMA,
            pltpu.SemaphoreType.DMA,
        ),
    )


def _make_s2(H):
    def body(exf, dstr, den0, den1,
             afo,
             dst_v, exb_f, d0b, d1b, ab_f, sem1, sem2):
        wid = _wid()
        iota = lax.iota(jnp.int32, 16)

        def chunk_body(t, carry):
            base = (wid * NCHUNK + t) * CH
            pltpu.sync_copy(dstr.at[pl.ds(base, CH)], dst_v)
            pltpu.sync_copy(exf.at[pl.ds(base * H, CH * H)], exb_f)
            cp1 = pltpu.async_copy(den0.at[dst_v], d0b, sem1)
            cp2 = pltpu.async_copy(den1.at[dst_v], d1b, sem2)
            cp1.wait()
            cp2.wait()
            for v in range((CH * H) // 16):
                p = iota + v * 16
                r = p >> 2
                cc = p & 3
                ev = exb_f[pl.ds(v * 16, 16)]
                d0 = plsc.load_gather(d0b, [r, cc])
                d1 = plsc.load_gather(d1b, [r, cc])
                ab_f[pl.ds(v * 16, 16)] = ev / (d0 + d1 + 1e-16)
            pltpu.sync_copy(ab_f, afo.at[pl.ds(base * H, CH * H)])
            return carry

        lax.fori_loop(0, NCHUNK, chunk_body, 0)

    return pl.kernel(
        body,
        out_type=(jax.ShapeDtypeStruct((EP * H,), jnp.float32),),
        mesh=_MESH,
        compiler_params=_SC_PARAMS,
        scratch_types=(
            pltpu.VMEM((CH,), jnp.int32),
            pltpu.VMEM((CH * H,), jnp.float32),
            pltpu.VMEM((CH, 16), jnp.float32),
            pltpu.VMEM((CH, 16), jnp.float32),
            pltpu.VMEM((CH * H,), jnp.float32),
            pltpu.SemaphoreType.DMA,
            pltpu.SemaphoreType.DMA,
        ),
    )


def _make_s3(H, C):
    D = H * C
    NBLK = D // 32

    def body(xlb, srcr, dstr, af, zr32,
             outp,
             src_v, dst_v, gidx_v, afb, xlr, wbuf, zb32, out_sh, sem1):
        c = lax.axis_index("c")
        s = lax.axis_index("s")
        wid = _wid()
        pltpu.sync_copy(zr32, zb32)
        for fb in range(NBLK):
            h = fb // (NBLK // H)
            for k in range(8):
                pltpu.sync_copy(
                    zb32, out_sh.at[pl.ds(s * RPT + k * ZROWS, ZROWS)])
            plsc.subcore_barrier()

            def chunk_body(t, carry):
                base = (wid * NCHUNK + t) * CH
                pltpu.sync_copy(srcr.at[pl.ds(base, CH)], src_v)
                pltpu.sync_copy(dstr.at[pl.ds(base, CH)], dst_v)
                pltpu.sync_copy(af.at[pl.ds(base * H, CH * H)], afb)
                for k in range(CH // 16):
                    gv = src_v[pl.ds(k * 16, 16)]
                    gidx_v[pl.ds(k * 16, 16)] = gv * NBLK + fb
                pltpu.async_copy(xlb.at[gidx_v], xlr, sem1).wait()

                def edge_body(e, ecarry):
                    a16 = plsc.load_gather(
                        afb, [jnp.broadcast_to(e * H + h, (16,))])
                    wbuf[e, pl.ds(0, 16)] = xlr[e, pl.ds(0, 16)] * a16
                    wbuf[e, pl.ds(16, 16)] = xlr[e, pl.ds(16, 16)] * a16
                    return ecarry

                lax.fori_loop(0, CH, edge_body, 0)
                pltpu.sync_copy(wbuf, out_sh.at[dst_v], add=True)
                return carry

            lax.fori_loop(0, NCHUNK, chunk_body, 0)
            plsc.subcore_barrier()

            @pl.when(c == 0)
            def _():
                pltpu.sync_copy(out_sh.at[pl.ds(s * RPT, RPT)],
                                outp.at[fb, 0, pl.ds(s * RPT, RPT)])

            @pl.when(c == 1)
            def _():
                pltpu.sync_copy(out_sh.at[pl.ds(s * RPT, RPT)],
                                outp.at[fb, 1, pl.ds(s * RPT, RPT)])

            plsc.subcore_barrier()

    def make(nblk):
        return pl.kernel(
            body,
            out_type=(jax.ShapeDtypeStruct((nblk, 2, NP, 32), jnp.float32),),
            mesh=_MESH,
            compiler_params=_SC_PARAMS,
            scratch_types=(
                pltpu.VMEM((CH,), jnp.int32),
                pltpu.VMEM((CH,), jnp.int32),
                pltpu.VMEM((CH,), jnp.int32),
                pltpu.VMEM((CH * H,), jnp.float32),
                pltpu.VMEM((CH, 32), jnp.float32),
                pltpu.VMEM((CH, 32), jnp.float32),
                pltpu.VMEM((ZROWS, 32), jnp.float32),
                pltpu.VMEM_SHARED((NP, 32), jnp.float32),
                pltpu.SemaphoreType.DMA,
            ),
        )

    return make(NBLK)


_S1_L1 = _make_s1(H1, C1)
_S1_L2 = _make_s1(H2, C2)
_S2 = _make_s2(4)
_S3_L1 = _make_s3(H1, C1)
_S3_L2 = _make_s3(H2, C2)


def _gat_layer_sc(s1, s3, xpad, src, dst, ea, We, att, bias, H, C):
    D = H * C
    wea = We.reshape(D)
    attf = att.reshape(D)
    zr4 = jnp.zeros((ZROWS, 16), jnp.float32)
    zr32 = jnp.zeros((ZROWS, 32), jnp.float32)
    xl = xpad[0]
    xr = xpad[1]
    ex, deno = s1(xl, xr, src, dst, ea, wea, attf, zr4)
    (af,) = _S2(ex.reshape(EP * H), dst, deno[:NP], deno[NP:])
    xlb = xl.reshape(NP * (D // 32), 32)
    (outp,) = s3(xlb, src, dst, af, zr32)
    out = outp.sum(axis=1)
    out = out.transpose(1, 0, 2).reshape(NP, D)[:N]
    return out + bias


def _bn_lrelu(h, g, b):
    mu = jnp.mean(h, axis=0)
    var = jnp.var(h, axis=0)
    return jax.nn.leaky_relu((h - mu) / jnp.sqrt(var + 1e-5) * g + b, 0.01)


def _mlp_head_kernel(h_ref, w1_ref, b1_ref, w2_ref, b2_ref, w3_ref, b3_ref, o_ref):
    h = h_ref[...]
    h = jax.nn.leaky_relu(h @ w1_ref[...] + b1_ref[...], 0.01)
    h = jax.nn.leaky_relu(h @ w2_ref[...] + b2_ref[...], 0.01)
    o_ref[...] = h @ w3_ref[...] + b3_ref[...]


def kernel(x, edge_index, edge_attr, batch, Wl1, bl1, Wr1, br1, We1, att1, bias1, bn1_g, bn1_b, Wl2, bl2, Wr2, br2, We2, att2, bias2, bn2_g, bn2_b, fc1_w, fc1_b, fc2_w, fc2_b, fc3_w, fc3_b):
    n = x.shape[0]
    loop = jnp.arange(n, dtype=jnp.int32)
    pad_e = EP - (E_RAW + n)
    src = jnp.concatenate(
        [edge_index[0], loop, jnp.zeros((pad_e,), jnp.int32)])
    dst = jnp.concatenate(
        [edge_index[1], loop, jnp.full((pad_e,), N, jnp.int32)])
    ea_mean = jnp.mean(edge_attr)
    ea = jnp.concatenate(
        [edge_attr[:, 0], jnp.full((n,), ea_mean, jnp.float32),
         jnp.zeros((pad_e,), jnp.float32)])

    def proj(h, W, b):
        out = h @ W + b
        return jnp.pad(out, ((0, NP - n), (0, 0)))

    xl1 = proj(x, Wl1, bl1)
    xr1 = proj(x, Wr1, br1)
    h = _gat_layer_sc(_S1_L1, _S3_L1, (xl1, xr1), src, dst, ea,
                      We1, att1, bias1, H1, C1)
    h = _bn_lrelu(h, bn1_g, bn1_b)
    xl2 = proj(h, Wl2, bl2)
    xr2 = proj(h, Wr2, br2)
    h = _gat_layer_sc(_S1_L2, _S3_L2, (xl2, xr2), src, dst, ea,
                      We2, att2, bias2, H2, C2)
    h = _bn_lrelu(h, bn2_g, bn2_b)
    onehot = (batch[:, None] == jnp.arange(B, dtype=batch.dtype)[None, :])
    onehot = onehot.astype(jnp.float32)
    s = onehot.T @ h
    cnt = jnp.sum(onehot, axis=0)
    h = s / jnp.maximum(cnt, 1.0)[:, None]
    out = pl.pallas_call(
        _mlp_head_kernel,
        out_shape=jax.ShapeDtypeStruct((B, 1), jnp.float32),
    )(h, fc1_w, fc1_b, fc2_w, fc2_b, fc3_w, fc3_b)
    return out


# speedup vs baseline: 13.8622x; 1.1969x over previous
import jax
import jax.numpy as jnp
from jax import lax
from jax.experimental import pallas as pl
from jax.experimental.pallas import tpu as pltpu
from jax.experimental.pallas import tpu_sc as plsc

N = 50000
B = 32
H1, C1 = 4, 32
H2, C2 = 4, 64

NP = 50048
RPT = NP // 16
E_RAW = 800000
EP = 851968
EPW = EP // 32
ZROWS = 391
ZROWS3 = 136

_MESH = plsc.VectorSubcoreMesh(core_axis_name="c", subcore_axis_name="s")
_SC_PARAMS = pltpu.CompilerParams(
    needs_layout_passes=False, use_tc_tiling_on_sc=False)


def _wid():
    return lax.axis_index("s") * 2 + lax.axis_index("c")


def _vcopy(src_ref, dst_ref, n16):
    for k in range(n16):
        dst_ref[pl.ds(k * 16, 16)] = src_ref[pl.ds(k * 16, 16)]


def _make_s1(H, C, CH):
    D = H * C
    CPH = C // 16
    NCH = EPW // CH

    def body(xl, xr, srcr, dstr, ear, wear, attr, zr4,
             exo, deno,
             src_v, dst_v, sidx, ea_v, xlr, xrr, exb, exb16, w_v, a_v, zb4,
             den_sh, gsem, ssem, osem):
        c = lax.axis_index("c")
        s = lax.axis_index("s")
        wid = _wid()
        pltpu.sync_copy(wear, w_v)
        pltpu.sync_copy(attr, a_v)
        pltpu.sync_copy(zr4, zb4)
        for k in range(8):
            pltpu.sync_copy(
                zb4, den_sh.at[pl.ds(s * RPT + k * ZROWS, ZROWS)])
        iota = lax.iota(jnp.int32, 16)
        lane15 = iota == 15
        for par in (0, 1):
            def zrow(g, zc, par=par):
                p = iota + g * 16
                plsc.store_scatter(exb16.at[par], [p >> 4, p & 15],
                                   jnp.zeros((16,), jnp.float32))
                return zc
            lax.fori_loop(0, CH, zrow, 0)
        plsc.subcore_barrier()

        def load_idx(t, par):
            base = wid * EPW + t * CH
            pltpu.sync_copy(srcr.at[pl.ds(base, CH)], src_v.at[par])
            pltpu.sync_copy(dstr.at[pl.ds(base, CH)], dst_v.at[par])
            pltpu.sync_copy(ear.at[pl.ds(base, CH)], ea_v.at[par])

        def fire_gather(par):
            pltpu.async_copy(xl.at[src_v.at[par]], xlr.at[par], gsem)
            pltpu.async_copy(xr.at[dst_v.at[par]], xrr.at[par], gsem)

        def wait_gather(par):
            pltpu.make_async_copy(xl.at[src_v.at[par]], xlr.at[par],
                                  gsem).wait()
            pltpu.make_async_copy(xr.at[dst_v.at[par]], xrr.at[par],
                                  gsem).wait()

        def drain_out(t, par):
            base2 = wid * EPW + t * CH
            pltpu.make_async_copy(
                exb.at[par], exo.at[pl.ds(base2, CH)], osem).wait()
            pltpu.make_async_copy(
                exb16.at[par], den_sh.at[sidx.at[par]], ssem).wait()

        load_idx(0, 0)
        fire_gather(0)

        def step(t, par):
            wait_gather(par)

            @pl.when(t + 1 < NCH)
            def _():
                load_idx(t + 1, 1 - par)
                fire_gather(1 - par)

            @pl.when(t >= 2)
            def _():
                drain_out(t - 2, par)

            def edge_body(e, ec):
                ea16 = plsc.load_gather(
                    ea_v.at[par], [jnp.broadcast_to(e, (16,))])
                for h in range(H):
                    acc = jnp.zeros((16,), jnp.float32)
                    for j2 in range(CPH):
                        j = h * CPH + j2
                        zl = xlr[par, e, pl.ds(j * 16, 16)]
                        zrv = xrr[par, e, pl.ds(j * 16, 16)]
                        wv = w_v[pl.ds(j * 16, 16)]
                        av = a_v[pl.ds(j * 16, 16)]
                        z = zl + zrv + ea16 * wv
                        m = jnp.maximum(z, 0.01 * z)
                        acc = acc + m * av
                    csum = plsc.cumsum(acc)
                    plsc.store_scatter(
                        exb.at[par],
                        [jnp.broadcast_to(e, (16,)),
                         jnp.full((16,), h, jnp.int32)],
                        csum, mask=lane15)
                return ec

            lax.fori_loop(0, CH, edge_body, 0)
            for v in range((CH * H) // 16):
                p = iota + v * 16
                r = p >> 2
                cc = p & 3
                val = plsc.load_gather(exb.at[par], [r, cc])
                ev = jnp.exp(val)
                plsc.store_scatter(exb.at[par], [r, cc], ev)
                plsc.store_scatter(exb16.at[par], [r, cc], ev)
            _vcopy(dst_v.at[par], sidx.at[par], CH // 16)
            base = wid * EPW + t * CH
            pltpu.async_copy(exb.at[par], exo.at[pl.ds(base, CH)], osem)
            pltpu.async_copy(exb16.at[par], den_sh.at[sidx.at[par]], ssem,
                             add=True)

        def two_steps(t2, carry):
            step(t2 * 2, 0)
            step(t2 * 2 + 1, 1)
            return carry

        lax.fori_loop(0, NCH // 2, two_steps, 0)
        for par in (0, 1):
            drain_out(NCH - 2 + par, par)
        plsc.subcore_barrier()
        pltpu.sync_copy(den_sh.at[pl.ds(s * RPT, RPT)],
                        deno.at[pl.ds(c * NP + s * RPT, RPT)])

    return pl.kernel(
        body,
        out_type=(
            jax.ShapeDtypeStruct((EP, H), jnp.float32),
            jax.ShapeDtypeStruct((2 * NP, 16), jnp.float32),
        ),
        mesh=_MESH,
        compiler_params=_SC_PARAMS,
        scratch_types=(
            pltpu.VMEM((2, CH), jnp.int32),
            pltpu.VMEM((2, CH), jnp.int32),
            pltpu.VMEM((2, CH), jnp.int32),
            pltpu.VMEM((2, CH), jnp.float32),
            pltpu.VMEM((2, CH, D), jnp.float32),
            pltpu.VMEM((2, CH, D), jnp.float32),
            pltpu.VMEM((2, CH, H), jnp.float32),
            pltpu.VMEM((2, CH, 16), jnp.float32),
            pltpu.VMEM((D,), jnp.float32),
            pltpu.VMEM((D,), jnp.float32),
            pltpu.VMEM((ZROWS, 16), jnp.float32),
            pltpu.VMEM_SHARED((NP, 16), jnp.float32),
            pltpu.SemaphoreType.DMA,
            pltpu.SemaphoreType.DMA,
            pltpu.SemaphoreType.DMA,
        ),
    )


def _make_s2(H, CH):
    NCH = EPW // CH

    def body(exf, dstr, den0, den1,
             afo,
             dst_v, exb_f, d0b, d1b, ab_f, sem1, sem2):
        wid = _wid()
        iota = lax.iota(jnp.int32, 16)

        def chunk_body(t, carry):
            base = wid * EPW + t * CH
            pltpu.sync_copy(dstr.at[pl.ds(base, CH)], dst_v)
            pltpu.sync_copy(exf.at[pl.ds(base * H, CH * H)], exb_f)
            cp1 = pltpu.async_copy(den0.at[dst_v], d0b, sem1)
            cp2 = pltpu.async_copy(den1.at[dst_v], d1b, sem2)
            cp1.wait()
            cp2.wait()
            for v in range((CH * H) // 16):
                p = iota + v * 16
                r = p >> 2
                cc = p & 3
                ev = exb_f[pl.ds(v * 16, 16)]
                d0 = plsc.load_gather(d0b, [r, cc])
                d1 = plsc.load_gather(d1b, [r, cc])
                ab_f[pl.ds(v * 16, 16)] = ev / (d0 + d1 + 1e-16)
            pltpu.sync_copy(ab_f, afo.at[pl.ds(base * H, CH * H)])
            return carry

        lax.fori_loop(0, NCH, chunk_body, 0)

    return pl.kernel(
        body,
        out_type=(jax.ShapeDtypeStruct((EP * H,), jnp.float32),),
        mesh=_MESH,
        compiler_params=_SC_PARAMS,
        scratch_types=(
            pltpu.VMEM((CH,), jnp.int32),
            pltpu.VMEM((CH * H,), jnp.float32),
            pltpu.VMEM((CH, 16), jnp.float32),
            pltpu.VMEM((CH, 16), jnp.float32),
            pltpu.VMEM((CH * H,), jnp.float32),
            pltpu.SemaphoreType.DMA,
            pltpu.SemaphoreType.DMA,
        ),
    )


def _make_s3(H, C, CH):
    D = H * C
    NBLK = D // 32
    NCH = EPW // CH

    def body(xlb, srcr, dstr, af, zr32,
             outp,
             src_v, dst_v, sidx, gidx_v, afb, xlr, wbuf, zb32, out_sh,
             gsem, ssem):
        c = lax.axis_index("c")
        s = lax.axis_index("s")
        wid = _wid()
        pltpu.sync_copy(zr32, zb32)
        for fb in range(NBLK):
            h = fb // (NBLK // H)
            for k in range(RPT // ZROWS3):
                pltpu.sync_copy(
                    zb32, out_sh.at[pl.ds(s * RPT + k * ZROWS3, ZROWS3)])
            plsc.subcore_barrier()

            def load_idx(t, par, fb=fb):
                base = wid * EPW + t * CH
                pltpu.sync_copy(srcr.at[pl.ds(base, CH)], src_v.at[par])
                pltpu.sync_copy(dstr.at[pl.ds(base, CH)], dst_v.at[par])
                pltpu.sync_copy(af.at[pl.ds(base * H, CH * H)],
                                afb.at[par])
                for k in range(CH // 16):
                    gv = src_v[par, pl.ds(k * 16, 16)]
                    gidx_v[par, pl.ds(k * 16, 16)] = gv * NBLK + fb

            def fire_gather(par):
                pltpu.async_copy(xlb.at[gidx_v.at[par]], xlr.at[par], gsem)

            def wait_gather(par):
                pltpu.make_async_copy(xlb.at[gidx_v.at[par]], xlr.at[par],
                                      gsem).wait()

            def drain_scatter(par):
                pltpu.make_async_copy(
                    wbuf.at[par], out_sh.at[sidx.at[par]], ssem).wait()

            load_idx(0, 0)
            fire_gather(0)

            def step(t, par, h=h):
                wait_gather(par)

                @pl.when(t + 1 < NCH)
                def _():
                    load_idx(t + 1, 1 - par)
                    fire_gather(1 - par)

                @pl.when(t >= 2)
                def _():
                    drain_scatter(par)

                def edge_body(e, ec):
                    a16 = plsc.load_gather(
                        afb.at[par], [jnp.broadcast_to(e * H + h, (16,))])
                    wbuf[par, e, pl.ds(0, 16)] = \
                        xlr[par, e, pl.ds(0, 16)] * a16
                    wbuf[par, e, pl.ds(16, 16)] = \
                        xlr[par, e, pl.ds(16, 16)] * a16
                    return ec

                lax.fori_loop(0, CH, edge_body, 0)
                _vcopy(dst_v.at[par], sidx.at[par], CH // 16)
                pltpu.async_copy(wbuf.at[par], out_sh.at[sidx.at[par]],
                                 ssem, add=True)

            def two_steps(t2, carry):
                step(t2 * 2, 0)
                step(t2 * 2 + 1, 1)
                return carry

            lax.fori_loop(0, NCH // 2, two_steps, 0)
            for par in (0, 1):
                drain_scatter(par)
            plsc.subcore_barrier()
            pltpu.sync_copy(out_sh.at[pl.ds(s * RPT, RPT)],
                            outp.at[fb, pl.ds(c * NP + s * RPT, RPT)])
            plsc.subcore_barrier()

    return pl.kernel(
        body,
        out_type=(jax.ShapeDtypeStruct((NBLK, 2 * NP, 32), jnp.float32),),
        mesh=_MESH,
        compiler_params=_SC_PARAMS,
        scratch_types=(
            pltpu.VMEM((2, CH), jnp.int32),
            pltpu.VMEM((2, CH), jnp.int32),
            pltpu.VMEM((2, CH), jnp.int32),
            pltpu.VMEM((2, CH), jnp.int32),
            pltpu.VMEM((2, CH * H), jnp.float32),
            pltpu.VMEM((2, CH, 32), jnp.float32),
            pltpu.VMEM((2, CH, 32), jnp.float32),
            pltpu.VMEM((ZROWS3, 32), jnp.float32),
            pltpu.VMEM_SHARED((NP, 32), jnp.float32),
            pltpu.SemaphoreType.DMA,
            pltpu.SemaphoreType.DMA,
        ),
    )


_S1_L1 = _make_s1(H1, C1, 128)
_S1_L2 = _make_s1(H2, C2, 64)
_S2 = _make_s2(4, 128)
_S3_L1 = _make_s3(H1, C1, 128)
_S3_L2 = _make_s3(H2, C2, 128)


def _gat_layer_sc(s1, s3, xl, xr, src, dst, ea, We, att, bias, H, C):
    D = H * C
    wea = We.reshape(D)
    attf = att.reshape(D)
    zr4 = jnp.zeros((ZROWS, 16), jnp.float32)
    zr32 = jnp.zeros((ZROWS3, 32), jnp.float32)
    ex, deno = s1(xl, xr, src, dst, ea, wea, attf, zr4)
    (af,) = _S2(ex.reshape(EP * H), dst, deno[:NP], deno[NP:])
    xlb = xl.reshape(NP * (D // 32), 32)
    (outp,) = s3(xlb, src, dst, af, zr32)
    out = outp[:, :NP] + outp[:, NP:]
    out = out.transpose(1, 0, 2).reshape(NP, D)[:N]
    return out + bias


def _bn_lrelu(h, g, b):
    mu = jnp.mean(h, axis=0)
    var = jnp.var(h, axis=0)
    return jax.nn.leaky_relu((h - mu) / jnp.sqrt(var + 1e-5) * g + b, 0.01)


def _mlp_head_kernel(h_ref, w1_ref, b1_ref, w2_ref, b2_ref, w3_ref, b3_ref, o_ref):
    h = h_ref[...]
    h = jax.nn.leaky_relu(h @ w1_ref[...] + b1_ref[...], 0.01)
    h = jax.nn.leaky_relu(h @ w2_ref[...] + b2_ref[...], 0.01)
    o_ref[...] = h @ w3_ref[...] + b3_ref[...]


def kernel(x, edge_index, edge_attr, batch, Wl1, bl1, Wr1, br1, We1, att1, bias1, bn1_g, bn1_b, Wl2, bl2, Wr2, br2, We2, att2, bias2, bn2_g, bn2_b, fc1_w, fc1_b, fc2_w, fc2_b, fc3_w, fc3_b):
    n = x.shape[0]
    loop = jnp.arange(n, dtype=jnp.int32)
    pad_e = EP - (E_RAW + n)
    src = jnp.concatenate(
        [edge_index[0], loop, jnp.zeros((pad_e,), jnp.int32)])
    dst = jnp.concatenate(
        [edge_index[1], loop, jnp.full((pad_e,), N, jnp.int32)])
    ea_mean = jnp.mean(edge_attr)
    ea = jnp.concatenate(
        [edge_attr[:, 0], jnp.full((n,), ea_mean, jnp.float32),
         jnp.zeros((pad_e,), jnp.float32)])

    def proj(h, W, b):
        out = h @ W + b
        return jnp.pad(out, ((0, NP - n), (0, 0)))

    xl1 = proj(x, Wl1, bl1)
    xr1 = proj(x, Wr1, br1)
    h = _gat_layer_sc(_S1_L1, _S3_L1, xl1, xr1, src, dst, ea,
                      We1, att1, bias1, H1, C1)
    h = _bn_lrelu(h, bn1_g, bn1_b)
    xl2 = proj(h, Wl2, bl2)
    xr2 = proj(h, Wr2, br2)
    h = _gat_layer_sc(_S1_L2, _S3_L2, xl2, xr2, src, dst, ea,
                      We2, att2, bias2, H2, C2)
    h = _bn_lrelu(h, bn2_g, bn2_b)
    onehot = (batch[:, None] == jnp.arange(B, dtype=batch.dtype)[None, :])
    onehot = onehot.astype(jnp.float32)
    s = onehot.T @ h
    cnt = jnp.sum(onehot, axis=0)
    h = s / jnp.maximum(cnt, 1.0)[:, None]
    out = pl.pallas_call(
        _mlp_head_kernel,
        out_shape=jax.ShapeDtypeStruct((B, 1), jnp.float32),
    )(h, fc1_w, fc1_b, fc2_w, fc2_b, fc3_w, fc3_b)
    return out


# speedup vs baseline: 16.6539x; 1.2014x over previous
import jax
import jax.numpy as jnp
from jax import lax
from jax.experimental import pallas as pl
from jax.experimental.pallas import tpu as pltpu
from jax.experimental.pallas import tpu_sc as plsc

N = 50000
B = 32
H1, C1 = 4, 32
H2, C2 = 4, 64

NP = 50048
RPT = NP // 16
E_RAW = 800000
EP = 851968
EPW = EP // 32
ZROWS = 391
ZROWS3 = 136

_MESH = plsc.VectorSubcoreMesh(core_axis_name="c", subcore_axis_name="s")
_SC_PARAMS = pltpu.CompilerParams(
    needs_layout_passes=False, use_tc_tiling_on_sc=False)


def _wid():
    return lax.axis_index("s") * 2 + lax.axis_index("c")


def _vcopy(src_ref, dst_ref, n16):
    for k in range(n16):
        dst_ref[pl.ds(k * 16, 16)] = src_ref[pl.ds(k * 16, 16)]


def _make_s1(H, C, CH):
    D = H * C
    CPH = C // 16
    NCH = EPW // CH

    def body(xl, xr, srcr, dstr, ear, wear, attr, zr4,
             exo, deno,
             src_v, dst_v, sidx, ea_v, xlr, xrr, exb, exb16, w_v, a_v, zb4,
             den_sh, gsem, ssem, osem, isem):
        c = lax.axis_index("c")
        s = lax.axis_index("s")
        wid = _wid()
        pltpu.sync_copy(wear, w_v)
        pltpu.sync_copy(attr, a_v)
        pltpu.sync_copy(zr4, zb4)
        for k in range(8):
            pltpu.sync_copy(
                zb4, den_sh.at[pl.ds(s * RPT + k * ZROWS, ZROWS)])
        iota = lax.iota(jnp.int32, 16)
        lane15 = iota == 15
        for par in (0, 1):
            def zrow(g, zc, par=par):
                p = iota + g * 16
                plsc.store_scatter(exb16.at[par], [p >> 4, p & 15],
                                   jnp.zeros((16,), jnp.float32))
                return zc
            lax.fori_loop(0, CH, zrow, 0)
        plsc.subcore_barrier()

        def load_idx(t, par):
            base = wid * EPW + t * CH
            c1 = pltpu.async_copy(srcr.at[pl.ds(base, CH)], src_v.at[par],
                                  isem)
            c2 = pltpu.async_copy(dstr.at[pl.ds(base, CH)], dst_v.at[par],
                                  isem)
            c3 = pltpu.async_copy(ear.at[pl.ds(base, CH)], ea_v.at[par],
                                  isem)
            c1.wait()
            c2.wait()
            c3.wait()

        def fire_gather(par):
            pltpu.async_copy(xl.at[src_v.at[par]], xlr.at[par], gsem)
            pltpu.async_copy(xr.at[dst_v.at[par]], xrr.at[par], gsem)

        def wait_gather(par):
            pltpu.make_async_copy(xl.at[src_v.at[par]], xlr.at[par],
                                  gsem).wait()
            pltpu.make_async_copy(xr.at[dst_v.at[par]], xrr.at[par],
                                  gsem).wait()

        def drain_out(t, par):
            base2 = wid * EPW + t * CH
            pltpu.make_async_copy(
                exb.at[par], exo.at[pl.ds(base2, CH)], osem).wait()
            pltpu.make_async_copy(
                exb16.at[par], den_sh.at[sidx.at[par]], ssem).wait()

        load_idx(0, 0)
        fire_gather(0)

        def step(t, par):
            wait_gather(par)

            @pl.when(t + 1 < NCH)
            def _():
                load_idx(t + 1, 1 - par)
                fire_gather(1 - par)

            @pl.when(t >= 2)
            def _():
                drain_out(t - 2, par)

            def edge_body(e, ec):
                ea16 = plsc.load_gather(
                    ea_v.at[par], [jnp.broadcast_to(e, (16,))])
                for h in range(H):
                    acc = jnp.zeros((16,), jnp.float32)
                    for j2 in range(CPH):
                        j = h * CPH + j2
                        zl = xlr[par, e, pl.ds(j * 16, 16)]
                        zrv = xrr[par, e, pl.ds(j * 16, 16)]
                        wv = w_v[pl.ds(j * 16, 16)]
                        av = a_v[pl.ds(j * 16, 16)]
                        z = zl + zrv + ea16 * wv
                        m = jnp.maximum(z, 0.01 * z)
                        acc = acc + m * av
                    csum = plsc.cumsum(acc)
                    plsc.store_scatter(
                        exb.at[par],
                        [jnp.broadcast_to(e, (16,)),
                         jnp.full((16,), h, jnp.int32)],
                        csum, mask=lane15)
                return ec

            lax.fori_loop(0, CH, edge_body, 0)
            for v in range((CH * H) // 16):
                p = iota + v * 16
                r = p >> 2
                cc = p & 3
                val = plsc.load_gather(exb.at[par], [r, cc])
                ev = jnp.exp(val)
                plsc.store_scatter(exb.at[par], [r, cc], ev)
                plsc.store_scatter(exb16.at[par], [r, cc], ev)
            _vcopy(dst_v.at[par], sidx.at[par], CH // 16)
            base = wid * EPW + t * CH
            pltpu.async_copy(exb.at[par], exo.at[pl.ds(base, CH)], osem)
            pltpu.async_copy(exb16.at[par], den_sh.at[sidx.at[par]], ssem,
                             add=True)

        def two_steps(t2, carry):
            step(t2 * 2, 0)
            step(t2 * 2 + 1, 1)
            return carry

        lax.fori_loop(0, NCH // 2, two_steps, 0)
        for par in (0, 1):
            drain_out(NCH - 2 + par, par)
        plsc.subcore_barrier()
        pltpu.sync_copy(den_sh.at[pl.ds(s * RPT, RPT)],
                        deno.at[pl.ds(c * NP + s * RPT, RPT)])

    return pl.kernel(
        body,
        out_type=(
            jax.ShapeDtypeStruct((EP, H), jnp.float32),
            jax.ShapeDtypeStruct((2 * NP, 16), jnp.float32),
        ),
        mesh=_MESH,
        compiler_params=_SC_PARAMS,
        scratch_types=(
            pltpu.VMEM((2, CH), jnp.int32),
            pltpu.VMEM((2, CH), jnp.int32),
            pltpu.VMEM((2, CH), jnp.int32),
            pltpu.VMEM((2, CH), jnp.float32),
            pltpu.VMEM((2, CH, D), jnp.float32),
            pltpu.VMEM((2, CH, D), jnp.float32),
            pltpu.VMEM((2, CH, H), jnp.float32),
            pltpu.VMEM((2, CH, 16), jnp.float32),
            pltpu.VMEM((D,), jnp.float32),
            pltpu.VMEM((D,), jnp.float32),
            pltpu.VMEM((ZROWS, 16), jnp.float32),
            pltpu.VMEM_SHARED((NP, 16), jnp.float32),
            pltpu.SemaphoreType.DMA,
            pltpu.SemaphoreType.DMA,
            pltpu.SemaphoreType.DMA,
            pltpu.SemaphoreType.DMA,
        ),
    )


def _make_s2(H, CH):
    NCH = EPW // CH

    def body(exf, dstr, den0, den1,
             afo,
             dst_v, exb_f, d0b, d1b, ab_f, sem1, sem2):
        wid = _wid()
        iota = lax.iota(jnp.int32, 16)

        def chunk_body(t, carry):
            base = wid * EPW + t * CH
            pltpu.sync_copy(dstr.at[pl.ds(base, CH)], dst_v)
            pltpu.sync_copy(exf.at[pl.ds(base * H, CH * H)], exb_f)
            cp1 = pltpu.async_copy(den0.at[dst_v], d0b, sem1)
            cp2 = pltpu.async_copy(den1.at[dst_v], d1b, sem2)
            cp1.wait()
            cp2.wait()
            for v in range((CH * H) // 16):
                p = iota + v * 16
                r = p >> 2
                cc = p & 3
                ev = exb_f[pl.ds(v * 16, 16)]
                d0 = plsc.load_gather(d0b, [r, cc])
                d1 = plsc.load_gather(d1b, [r, cc])
                ab_f[pl.ds(v * 16, 16)] = ev / (d0 + d1 + 1e-16)
            pltpu.sync_copy(ab_f, afo.at[pl.ds(base * H, CH * H)])
            return carry

        lax.fori_loop(0, NCH, chunk_body, 0)

    return pl.kernel(
        body,
        out_type=(jax.ShapeDtypeStruct((EP * H,), jnp.float32),),
        mesh=_MESH,
        compiler_params=_SC_PARAMS,
        scratch_types=(
            pltpu.VMEM((CH,), jnp.int32),
            pltpu.VMEM((CH * H,), jnp.float32),
            pltpu.VMEM((CH, 16), jnp.float32),
            pltpu.VMEM((CH, 16), jnp.float32),
            pltpu.VMEM((CH * H,), jnp.float32),
            pltpu.SemaphoreType.DMA,
            pltpu.SemaphoreType.DMA,
        ),
    )


def _make_s3(H, C, CH):
    D = H * C
    NBLK = D // 32
    NCH = EPW // CH

    def body(xlb, srcr, dstr, af, zr32,
             outp,
             src_v, dst_v, sidx, gidx_v, afb, xlr, wbuf, zb32, out_sh,
             gsem, ssem, isem, asem):
        c = lax.axis_index("c")
        s = lax.axis_index("s")
        wid = _wid()
        pltpu.sync_copy(zr32, zb32)
        for fb in range(NBLK):
            h = fb // (NBLK // H)
            for k in range(RPT // ZROWS3):
                pltpu.sync_copy(
                    zb32, out_sh.at[pl.ds(s * RPT + k * ZROWS3, ZROWS3)])
            plsc.subcore_barrier()

            def load_idx(t, par, fb=fb):
                base = wid * EPW + t * CH
                c1 = pltpu.async_copy(srcr.at[pl.ds(base, CH)],
                                      src_v.at[par], isem)
                c2 = pltpu.async_copy(dstr.at[pl.ds(base, CH)],
                                      dst_v.at[par], isem)
                c3 = pltpu.async_copy(af.at[pl.ds(base * H, CH * H)],
                                      afb.at[par], asem)
                c1.wait()
                c2.wait()
                c3.wait()
                for k in range(CH // 16):
                    gv = src_v[par, pl.ds(k * 16, 16)]
                    gidx_v[par, pl.ds(k * 16, 16)] = gv * NBLK + fb

            def fire_gather(par):
                pltpu.async_copy(xlb.at[gidx_v.at[par]], xlr.at[par], gsem)

            def wait_gather(par):
                pltpu.make_async_copy(xlb.at[gidx_v.at[par]], xlr.at[par],
                                      gsem).wait()

            def drain_scatter(par):
                pltpu.make_async_copy(
                    wbuf.at[par], out_sh.at[sidx.at[par]], ssem).wait()

            load_idx(0, 0)
            fire_gather(0)

            def step(t, par, h=h):
                wait_gather(par)

                @pl.when(t + 1 < NCH)
                def _():
                    load_idx(t + 1, 1 - par)
                    fire_gather(1 - par)

                @pl.when(t >= 2)
                def _():
                    drain_scatter(par)

                def edge_body(e, ec):
                    a16 = plsc.load_gather(
                        afb.at[par], [jnp.broadcast_to(e * H + h, (16,))])
                    wbuf[par, e, pl.ds(0, 16)] = \
                        xlr[par, e, pl.ds(0, 16)] * a16
                    wbuf[par, e, pl.ds(16, 16)] = \
                        xlr[par, e, pl.ds(16, 16)] * a16
                    return ec

                lax.fori_loop(0, CH, edge_body, 0)
                _vcopy(dst_v.at[par], sidx.at[par], CH // 16)
                pltpu.async_copy(wbuf.at[par], out_sh.at[sidx.at[par]],
                                 ssem, add=True)

            def two_steps(t2, carry):
                step(t2 * 2, 0)
                step(t2 * 2 + 1, 1)
                return carry

            lax.fori_loop(0, NCH // 2, two_steps, 0)
            for par in (0, 1):
                drain_scatter(par)
            plsc.subcore_barrier()
            pltpu.sync_copy(out_sh.at[pl.ds(s * RPT, RPT)],
                            outp.at[fb, pl.ds(c * NP + s * RPT, RPT)])
            plsc.subcore_barrier()

    return pl.kernel(
        body,
        out_type=(jax.ShapeDtypeStruct((NBLK, 2 * NP, 32), jnp.float32),),
        mesh=_MESH,
        compiler_params=_SC_PARAMS,
        scratch_types=(
            pltpu.VMEM((2, CH), jnp.int32),
            pltpu.VMEM((2, CH), jnp.int32),
            pltpu.VMEM((2, CH), jnp.int32),
            pltpu.VMEM((2, CH), jnp.int32),
            pltpu.VMEM((2, CH * H), jnp.float32),
            pltpu.VMEM((2, CH, 32), jnp.float32),
            pltpu.VMEM((2, CH, 32), jnp.float32),
            pltpu.VMEM((ZROWS3, 32), jnp.float32),
            pltpu.VMEM_SHARED((NP, 32), jnp.float32),
            pltpu.SemaphoreType.DMA,
            pltpu.SemaphoreType.DMA,
            pltpu.SemaphoreType.DMA,
            pltpu.SemaphoreType.DMA,
        ),
    )


_S1_L1 = _make_s1(H1, C1, 128)
_S1_L2 = _make_s1(H2, C2, 64)
_S2 = _make_s2(4, 128)
_S3_L1 = _make_s3(H1, C1, 128)
_S3_L2 = _make_s3(H2, C2, 128)


def _gat_layer_sc(s1, s3, xl, xr, src, dst, ea, We, att, bias, H, C):
    D = H * C
    wea = We.reshape(D)
    attf = att.reshape(D)
    zr4 = jnp.zeros((ZROWS, 16), jnp.float32)
    zr32 = jnp.zeros((ZROWS3, 32), jnp.float32)
    ex, deno = s1(xl, xr, src, dst, ea, wea, attf, zr4)
    (af,) = _S2(ex.reshape(EP * H), dst, deno[:NP], deno[NP:])
    xlb = xl.reshape(NP * (D // 32), 32)
    (outp,) = s3(xlb, src, dst, af, zr32)
    out = outp[:, :NP] + outp[:, NP:]
    out = out.transpose(1, 0, 2).reshape(NP, D)[:N]
    return out + bias


def _bn_lrelu(h, g, b):
    mu = jnp.mean(h, axis=0)
    var = jnp.var(h, axis=0)
    return jax.nn.leaky_relu((h - mu) / jnp.sqrt(var + 1e-5) * g + b, 0.01)


def _mlp_head_kernel(h_ref, w1_ref, b1_ref, w2_ref, b2_ref, w3_ref, b3_ref, o_ref):
    h = h_ref[...]
    h = jax.nn.leaky_relu(h @ w1_ref[...] + b1_ref[...], 0.01)
    h = jax.nn.leaky_relu(h @ w2_ref[...] + b2_ref[...], 0.01)
    o_ref[...] = h @ w3_ref[...] + b3_ref[...]


def kernel(x, edge_index, edge_attr, batch, Wl1, bl1, Wr1, br1, We1, att1, bias1, bn1_g, bn1_b, Wl2, bl2, Wr2, br2, We2, att2, bias2, bn2_g, bn2_b, fc1_w, fc1_b, fc2_w, fc2_b, fc3_w, fc3_b):
    n = x.shape[0]
    loop = jnp.arange(n, dtype=jnp.int32)
    pad_e = EP - (E_RAW + n)
    src = jnp.concatenate(
        [edge_index[0], loop, jnp.zeros((pad_e,), jnp.int32)])
    dst = jnp.concatenate(
        [edge_index[1], loop, jnp.full((pad_e,), N, jnp.int32)])
    ea_mean = jnp.mean(edge_attr)
    ea = jnp.concatenate(
        [edge_attr[:, 0], jnp.full((n,), ea_mean, jnp.float32),
         jnp.zeros((pad_e,), jnp.float32)])

    def proj(h, W, b):
        out = h @ W + b
        return jnp.pad(out, ((0, NP - n), (0, 0)))

    xl1 = proj(x, Wl1, bl1)
    xr1 = proj(x, Wr1, br1)
    h = _gat_layer_sc(_S1_L1, _S3_L1, xl1, xr1, src, dst, ea,
                      We1, att1, bias1, H1, C1)
    h = _bn_lrelu(h, bn1_g, bn1_b)
    xl2 = proj(h, Wl2, bl2)
    xr2 = proj(h, Wr2, br2)
    h = _gat_layer_sc(_S1_L2, _S3_L2, xl2, xr2, src, dst, ea,
                      We2, att2, bias2, H2, C2)
    h = _bn_lrelu(h, bn2_g, bn2_b)
    onehot = (batch[:, None] == jnp.arange(B, dtype=batch.dtype)[None, :])
    onehot = onehot.astype(jnp.float32)
    s = onehot.T @ h
    cnt = jnp.sum(onehot, axis=0)
    h = s / jnp.maximum(cnt, 1.0)[:, None]
    out = pl.pallas_call(
        _mlp_head_kernel,
        out_shape=jax.ShapeDtypeStruct((B, 1), jnp.float32),
    )(h, fc1_w, fc1_b, fc2_w, fc2_b, fc3_w, fc3_b)
    return out
